# baseline (speedup 1.0000x reference)
import functools

import jax
import jax.numpy as jnp
from jax import lax
from jax.experimental import pallas as pl
from jax.experimental.pallas import tpu as pltpu
from jax.experimental.pallas import tpu_sc as plsc

_INFO = plsc.get_sparse_core_info()
_NC = _INFO.num_cores
_NS = _INFO.num_subcores
_NW = _NC * _NS

_CHUNK = 16


def _sc_body(n_chunks, batch, w_hbm, out_hbm, buf0, buf1, sem0, sem1, wsem):
    wid = lax.axis_index("s") * _NC + lax.axis_index("c")
    base = wid * (n_chunks * _CHUNK)

    bufs = (buf0, buf1)
    in_sems = (sem0, sem1)

    def gather(k):
        pltpu.async_copy(
            w_hbm.at[pl.ds(base + k * _CHUNK, _CHUNK), :],
            bufs[k % 2], in_sems[k % 2],
        )

    gather(0)
    for k in range(n_chunks):
        slot = k % 2
        pltpu.make_async_copy(
            w_hbm.at[pl.ds(base + k * _CHUNK, _CHUNK), :],
            bufs[slot], in_sems[slot],
        ).wait()
        if k + 1 < n_chunks:
            gather(k + 1)
        r = base + k * _CHUNK
        for b in range(batch):
            pltpu.async_copy(bufs[slot], out_hbm.at[b, pl.ds(r, _CHUNK), :], wsem)
        for b in range(batch):
            pltpu.make_async_copy(
                bufs[slot], out_hbm.at[b, pl.ds(r, _CHUNK), :], wsem
            ).wait()


def kernel(tokens, W_pos):
    B, S = tokens.shape
    D = W_pos.shape[1]
    assert S % (_NW * _CHUNK) == 0
    n_chunks = S // (_NW * _CHUNK)

    mesh = plsc.VectorSubcoreMesh(core_axis_name="c", subcore_axis_name="s")
    k = functools.partial(
        pl.kernel,
        mesh=mesh,
        out_type=jax.ShapeDtypeStruct((B, S, D), jnp.float32),
        scratch_types=[
            pltpu.VMEM((_CHUNK, D), jnp.float32),
            pltpu.VMEM((_CHUNK, D), jnp.float32),
            pltpu.SemaphoreType.DMA,
            pltpu.SemaphoreType.DMA,
            pltpu.SemaphoreType.DMA,
        ],
    )(functools.partial(_sc_body, n_chunks, B))
    return k(W_pos)


# speedup vs baseline: 1.3294x; 1.3294x over previous
import jax
import jax.numpy as jnp
from jax.experimental import pallas as pl


def _copy_body(w_ref, o_ref):
    o_ref[0] = w_ref[...]


def kernel(tokens, W_pos):
    B, S = tokens.shape
    D = W_pos.shape[1]
    BS = 1024

    grid = (S // BS, B)
    out = pl.pallas_call(
        _copy_body,
        grid=grid,
        in_specs=[pl.BlockSpec((BS, D), lambda i, b: (i, 0))],
        out_specs=pl.BlockSpec((1, BS, D), lambda i, b: (b, i, 0)),
        out_shape=jax.ShapeDtypeStruct((B, S, D), jnp.float32),
    )(W_pos)
    return out


# speedup vs baseline: 1.4574x; 1.0964x over previous
import jax
import jax.numpy as jnp
from jax.experimental import pallas as pl


def _copy_body(w_ref, o_ref):
    o_ref[...] = jnp.broadcast_to(w_ref[...], o_ref.shape)


def kernel(tokens, W_pos):
    B, S = tokens.shape
    D = W_pos.shape[1]
    BS = 1024

    grid = (S // BS, B // 2)
    out = pl.pallas_call(
        _copy_body,
        grid=grid,
        in_specs=[pl.BlockSpec((BS, D), lambda i, b: (i, 0))],
        out_specs=pl.BlockSpec((2, BS, D), lambda i, b: (b, i, 0)),
        out_shape=jax.ShapeDtypeStruct((B, S, D), jnp.float32),
    )(W_pos)
    return out


# speedup vs baseline: 1.4792x; 1.0149x over previous
import jax
import jax.numpy as jnp
from jax.experimental import pallas as pl


def _copy_body(w_ref, o_ref):
    o_ref[...] = jnp.broadcast_to(w_ref[...], o_ref.shape)


def kernel(tokens, W_pos):
    B, S = tokens.shape
    D = W_pos.shape[1]
    BS = 512

    grid = (S // BS,)
    out = pl.pallas_call(
        _copy_body,
        grid=grid,
        in_specs=[pl.BlockSpec((BS, D), lambda i: (i, 0))],
        out_specs=pl.BlockSpec((B, BS, D), lambda i: (0, i, 0)),
        out_shape=jax.ShapeDtypeStruct((B, S, D), jnp.float32),
    )(W_pos)
    return out
